# baseline (speedup 1.0000x reference)
import jax
import jax.numpy as jnp
from jax.experimental import pallas as pl
from jax.experimental.pallas import tpu as pltpu

_H = 384
_W = 384
_N = float(_H * _W)
_R = _H // 8
_CPB = 8
_NB = 96 // _CPB


def _top10_sum_fallback(xs_ref, pc, tau):
    def step(_, carry):
        bound, acc, rem = carry
        v = xs_ref[pc]
        w = jnp.where((v >= tau) & (v < bound), v, -jnp.inf)
        mx = jnp.max(w)
        cc = jnp.sum(jnp.where(w == mx, 1.0, 0.0))
        take = jnp.minimum(cc, rem)
        acc = acc + jnp.where(take > 0.0, take * mx, 0.0)
        rem = rem - take
        return mx, acc, rem

    _, acc, _ = jax.lax.fori_loop(
        0, 10, step,
        (jnp.float32(jnp.inf), jnp.float32(0.0), jnp.float32(10.0))
    )
    return acc


def _body(x_ref, gt_ref, m_ref, out_ref,
          m2_ref, m2g_ref, cef_ref, xs_ref, wsv_ref, acc_ref):
    i = pl.program_id(0)
    par = jax.lax.rem(i, 2)
    prev = jax.lax.rem(i + 1, 2)

    @pl.when(i == 0)
    def _():
        m = m_ref[0]
        g = gt_ref[0]
        m2 = m * m
        m2g = m2 * g
        m2_ref[0] = m2
        m2g_ref[0] = m2g
        cef_ref[0] = jnp.sum(m2)
        cef_ref[1] = jnp.sum(m2g)
        cef_ref[2] = jnp.sum(m2g * g)
        out_ref[...] = jnp.zeros(out_ref.shape, jnp.float32)

    def chunk(j, carry):
        m2c = m2_ref[0, j]
        m2gc = m2g_ref[0, j]
        accs = []
        for c in range(_CPB):
            aS, aA, aB, aD, aM = carry[c]
            xv = x_ref[c, j]
            xs_ref[par * _CPB + c, j] = xv
            vm2 = xv * m2c
            aS = aS + xv
            aA = aA + xv * vm2
            aB = aB + vm2
            aD = aD + xv * m2gc
            aM = jnp.maximum(aM, xv)
            accs.append((aS, aA, aB, aD, aM))
        return tuple(accs)

    zero = jnp.zeros((8, _W), jnp.float32)
    init = tuple(
        (zero, zero, zero, zero,
         jnp.full((8, _W), -jnp.inf, jnp.float32))
        for _ in range(_CPB)
    )
    final = jax.lax.fori_loop(0, _R, chunk, init, unroll=True)

    def fold3(a):
        return jnp.maximum(jnp.maximum(a[:, :128], a[:, 128:256]),
                           a[:, 256:])

    def fold3s(a):
        return a[:, :128] + a[:, 128:256] + a[:, 256:]

    for c in range(_CPB):
        aS, aA, aB, aD, M = final[c]
        wsv_ref[par * _CPB + c] = fold3(M)
        acc_ref[par * _CPB + c, 0] = fold3s(aS)
        acc_ref[par * _CPB + c, 1] = fold3s(aA)
        acc_ref[par * _CPB + c, 2] = fold3s(aB)
        acc_ref[par * _CPB + c, 3] = fold3s(aD)

    Ws = [wsv_ref[prev * _CPB + c] for c in range(_CPB)]

    def tau_step(_, carry):
        out = []
        for c in range(_CPB):
            W, _tau = carry[c]
            mx = jnp.max(W, axis=1, keepdims=True)
            mx = jnp.max(mx, axis=0, keepdims=True)
            mxb = jax.lax.broadcast_in_dim(mx, (8, 128), (0, 1))
            W = jnp.where(W == mxb, -jnp.inf, W)
            out.append((W, mxb))
        return tuple(out)

    taus_c = jax.lax.fori_loop(
        0, 10, tau_step,
        tuple((Ws[c], Ws[c]) for c in range(_CPB)),
        unroll=True,
    )
    tau_wide = [
        jnp.concatenate([taus_c[c][1]] * (_W // 128), axis=1)
        for c in range(_CPB)
    ]

    def fchunk(j, carry):
        out = []
        for c in range(_CPB):
            aC, aV = carry[c]
            xv = xs_ref[prev * _CPB + c, j]
            sel = xv >= tau_wide[c]
            aC = aC + jnp.where(sel, 1.0, 0.0)
            aV = aV + jnp.where(sel, xv, 0.0)
            out.append((aC, aV))
        return tuple(out)

    facc = jax.lax.fori_loop(
        0, _R, fchunk, tuple((zero, zero) for _ in range(_CPB)),
        unroll=True,
    )

    C = cef_ref[0]
    E = cef_ref[1]
    F = cef_ref[2]

    cnts = [jnp.sum(facc[c][0]) for c in range(_CPB)]
    ssums = [jnp.sum(facc[c][1]) for c in range(_CPB)]

    all_exact = (cnts[0] == 10.0)
    for c in range(1, _CPB):
        all_exact = all_exact & (cnts[c] == 10.0)
    all_exact = all_exact | (i == 0)

    def _common(_):
        return tuple(ssums)

    def _rare(_):
        out = []
        for c in range(_CPB):
            tau_s = taus_c[c][1][0, 0]
            out.append(jax.lax.cond(
                cnts[c] == 10.0, lambda _, cc=c: ssums[cc],
                lambda _, cc=c, ts=tau_s: _top10_sum_fallback(
                    xs_ref, prev * _CPB + cc, ts),
                operand=None))
        return tuple(out)

    top10_sums = jax.lax.cond(all_exact, _common, _rare, operand=None)

    loss = jnp.float32(0.0)
    for c in range(_CPB):
        top10_sum = top10_sums[c]
        S = jnp.sum(acc_ref[prev * _CPB + c, 0])
        A = jnp.sum(acc_ref[prev * _CPB + c, 1])
        B = jnp.sum(acc_ref[prev * _CPB + c, 2])
        D = jnp.sum(acc_ref[prev * _CPB + c, 3])
        mu = S / _N
        max_avg = top10_sum / 10.0 - mu
        denom = jnp.where(max_avg < 1e-20, max_avg + 1e-19, max_avg)
        num = ((A - 2.0 * mu * B + mu * mu * C) / denom
               - 2.0 * (D - mu * E)) / denom + F
        loss = loss + num / _N

    loss = jnp.where(i > 0, loss, 0.0)
    out_ref[...] += jnp.full(out_ref.shape, loss, dtype=jnp.float32)


@jax.jit
def kernel(pattern, pattern_gt, mask):
    ch = pattern.shape[1]
    x = pattern.reshape(ch, _R, 8, _W)
    out = pl.pallas_call(
        _body,
        grid=(_NB + 1,),
        in_specs=[
            pl.BlockSpec((_CPB, _R, 8, _W),
                         lambda i: (jnp.minimum(i, _NB - 1), 0, 0, 0)),
            pl.BlockSpec((1, _R, 8, _W), lambda i: (0, 0, 0, 0)),
            pl.BlockSpec((1, _R, 8, _W), lambda i: (0, 0, 0, 0)),
        ],
        out_specs=pl.BlockSpec((8, 128), lambda i: (0, 0)),
        out_shape=jax.ShapeDtypeStruct((8, 128), jnp.float32),
        scratch_shapes=[
            pltpu.VMEM((1, _R, 8, _W), jnp.float32),
            pltpu.VMEM((1, _R, 8, _W), jnp.float32),
            pltpu.SMEM((3,), jnp.float32),
            pltpu.VMEM((2 * _CPB, _R, 8, _W), jnp.float32),
            pltpu.VMEM((2 * _CPB, 8, 128), jnp.float32),
            pltpu.VMEM((2 * _CPB, 4, 8, 128), jnp.float32),
        ],
    )(x, pattern_gt.reshape(1, _R, 8, _W), mask.reshape(1, _R, 8, _W))
    return out[0, 0].reshape(1)


# speedup vs baseline: 1.0912x; 1.0912x over previous
import jax
import jax.numpy as jnp
from jax.experimental import pallas as pl
from jax.experimental.pallas import tpu as pltpu

_H = 384
_W = 384
_N = float(_H * _W)
_R = _H // 8
_CPB = 8
_NB = 96 // _CPB


def _top10_sum_fallback(xs_ref, c, tau):
    def step(_, carry):
        bound, acc, rem = carry
        v = xs_ref[c]
        w = jnp.where((v >= tau) & (v < bound), v, -jnp.inf)
        mx = jnp.max(w)
        cc = jnp.sum(jnp.where(w == mx, 1.0, 0.0))
        take = jnp.minimum(cc, rem)
        acc = acc + jnp.where(take > 0.0, take * mx, 0.0)
        rem = rem - take
        return mx, acc, rem

    _, acc, _ = jax.lax.fori_loop(
        0, 10, step,
        (jnp.float32(jnp.inf), jnp.float32(0.0), jnp.float32(10.0))
    )
    return acc


def _fold3(a):
    return jnp.maximum(jnp.maximum(a[:, :128], a[:, 128:256]), a[:, 256:])


def _fold3s(a):
    return a[:, :128] + a[:, 128:256] + a[:, 256:]


def _main_part(x_ref, m2_ref, m2g_ref, xs_ref, wsv_ref, acc_ref):
    def chunk(j, carry):
        m2c = m2_ref[0, j]
        m2gc = m2g_ref[0, j]
        accs = []
        for c in range(_CPB):
            aS, aA, aB, aD, aM = carry[c]
            xv = x_ref[c, j]
            xs_ref[c, j] = xv
            vm2 = xv * m2c
            aS = aS + xv
            aA = aA + xv * vm2
            aB = aB + vm2
            aD = aD + xv * m2gc
            aM = jnp.maximum(aM, xv)
            accs.append((aS, aA, aB, aD, aM))
        return tuple(accs)

    zero = jnp.zeros((8, _W), jnp.float32)
    init = tuple(
        (zero, zero, zero, zero,
         jnp.full((8, _W), -jnp.inf, jnp.float32))
        for _ in range(_CPB)
    )
    final = jax.lax.fori_loop(0, _R, chunk, init, unroll=True)

    for c in range(_CPB):
        aS, aA, aB, aD, M = final[c]
        wsv_ref[c] = _fold3(M)
        acc_ref[c, 0] = _fold3s(aS)
        acc_ref[c, 1] = _fold3s(aA)
        acc_ref[c, 2] = _fold3s(aB)
        acc_ref[c, 3] = _fold3s(aD)


def _tail_part(i, xs_ref, wsv_ref, acc_ref, cef_ref, out_ref):
    Ws = [wsv_ref[c] for c in range(_CPB)]

    def tau_step(_, carry):
        out = []
        for c in range(_CPB):
            W, _tau = carry[c]
            mx = jnp.max(W, axis=1, keepdims=True)
            mx = jnp.max(mx, axis=0, keepdims=True)
            mxb = jax.lax.broadcast_in_dim(mx, (8, 128), (0, 1))
            W = jnp.where(W == mxb, -jnp.inf, W)
            out.append((W, mxb))
        return tuple(out)

    taus_c = jax.lax.fori_loop(
        0, 10, tau_step,
        tuple((Ws[c], Ws[c]) for c in range(_CPB)),
        unroll=True,
    )
    tau_wide = [
        jnp.concatenate([taus_c[c][1]] * (_W // 128), axis=1)
        for c in range(_CPB)
    ]

    def fchunk(j, carry):
        out = []
        for c in range(_CPB):
            aC, aV = carry[c]
            xv = xs_ref[c, j]
            sel = xv >= tau_wide[c]
            aC = aC + jnp.where(sel, 1.0, 0.0)
            aV = aV + jnp.where(sel, xv, 0.0)
            out.append((aC, aV))
        return tuple(out)

    zero = jnp.zeros((8, _W), jnp.float32)
    facc = jax.lax.fori_loop(
        0, _R, fchunk, tuple((zero, zero) for _ in range(_CPB)),
        unroll=True,
    )

    C = cef_ref[0]
    E = cef_ref[1]
    F = cef_ref[2]

    cnts = [jnp.sum(facc[c][0]) for c in range(_CPB)]
    ssums = [jnp.sum(facc[c][1]) for c in range(_CPB)]

    all_exact = (cnts[0] == 10.0)
    for c in range(1, _CPB):
        all_exact = all_exact & (cnts[c] == 10.0)
    all_exact = all_exact | (i == 0)

    def _common(_):
        return tuple(ssums)

    def _rare(_):
        out = []
        for c in range(_CPB):
            tau_s = taus_c[c][1][0, 0]
            out.append(jax.lax.cond(
                cnts[c] == 10.0, lambda _, cc=c: ssums[cc],
                lambda _, cc=c, ts=tau_s: _top10_sum_fallback(
                    xs_ref, cc, ts),
                operand=None))
        return tuple(out)

    top10_sums = jax.lax.cond(all_exact, _common, _rare, operand=None)

    loss = jnp.float32(0.0)
    for c in range(_CPB):
        top10_sum = top10_sums[c]
        S = jnp.sum(acc_ref[c, 0])
        A = jnp.sum(acc_ref[c, 1])
        B = jnp.sum(acc_ref[c, 2])
        D = jnp.sum(acc_ref[c, 3])
        mu = S / _N
        max_avg = top10_sum / 10.0 - mu
        denom = jnp.where(max_avg < 1e-20, max_avg + 1e-19, max_avg)
        num = ((A - 2.0 * mu * B + mu * mu * C) / denom
               - 2.0 * (D - mu * E)) / denom + F
        loss = loss + num / _N

    loss = jnp.where(i > 0, loss, 0.0)
    out_ref[...] += jnp.full(out_ref.shape, loss, dtype=jnp.float32)


def _body(x_ref, gt_ref, m_ref, out_ref,
          m2_ref, m2g_ref, cef_ref,
          xs0_ref, xs1_ref, wsv0_ref, wsv1_ref, acc0_ref, acc1_ref):
    i = pl.program_id(0)
    par = jax.lax.rem(i, 2)

    @pl.when(i == 0)
    def _():
        m = m_ref[0]
        g = gt_ref[0]
        m2 = m * m
        m2g = m2 * g
        m2_ref[0] = m2
        m2g_ref[0] = m2g
        cef_ref[0] = jnp.sum(m2)
        cef_ref[1] = jnp.sum(m2g)
        cef_ref[2] = jnp.sum(m2g * g)
        out_ref[...] = jnp.zeros(out_ref.shape, jnp.float32)

    @pl.when(par == 0)
    def _():
        _main_part(x_ref, m2_ref, m2g_ref, xs0_ref, wsv0_ref, acc0_ref)
        _tail_part(i, xs1_ref, wsv1_ref, acc1_ref, cef_ref, out_ref)

    @pl.when(par == 1)
    def _():
        _main_part(x_ref, m2_ref, m2g_ref, xs1_ref, wsv1_ref, acc1_ref)
        _tail_part(i, xs0_ref, wsv0_ref, acc0_ref, cef_ref, out_ref)


@jax.jit
def kernel(pattern, pattern_gt, mask):
    ch = pattern.shape[1]
    x = pattern.reshape(ch, _R, 8, _W)
    out = pl.pallas_call(
        _body,
        grid=(_NB + 1,),
        in_specs=[
            pl.BlockSpec((_CPB, _R, 8, _W),
                         lambda i: (jnp.minimum(i, _NB - 1), 0, 0, 0)),
            pl.BlockSpec((1, _R, 8, _W), lambda i: (0, 0, 0, 0)),
            pl.BlockSpec((1, _R, 8, _W), lambda i: (0, 0, 0, 0)),
        ],
        out_specs=pl.BlockSpec((8, 128), lambda i: (0, 0)),
        out_shape=jax.ShapeDtypeStruct((8, 128), jnp.float32),
        scratch_shapes=[
            pltpu.VMEM((1, _R, 8, _W), jnp.float32),
            pltpu.VMEM((1, _R, 8, _W), jnp.float32),
            pltpu.SMEM((3,), jnp.float32),
            pltpu.VMEM((_CPB, _R, 8, _W), jnp.float32),
            pltpu.VMEM((_CPB, _R, 8, _W), jnp.float32),
            pltpu.VMEM((_CPB, 8, 128), jnp.float32),
            pltpu.VMEM((_CPB, 8, 128), jnp.float32),
            pltpu.VMEM((_CPB, 4, 8, 128), jnp.float32),
            pltpu.VMEM((_CPB, 4, 8, 128), jnp.float32),
        ],
    )(x, pattern_gt.reshape(1, _R, 8, _W), mask.reshape(1, _R, 8, _W))
    return out[0, 0].reshape(1)
